# Initial kernel scaffold; baseline (speedup 1.0000x reference)
import jax
import jax.numpy as jnp
from jax.experimental import pallas as pl


def kernel(x, emb_weight):
    raise NotImplementedError("write your pallas kernel here")


# speedup vs baseline: 5.3337x; 5.3337x over previous
import functools

import jax
import jax.numpy as jnp
from jax import lax
from jax.experimental import pallas as pl
from jax.experimental.pallas import tpu as pltpu
from jax.experimental.pallas import tpu_sc as plsc

_ROWS, _COLS, _DIM = 16384, 200, 3
_N_IN = _ROWS * _COLS
_NC, _NS, _L = 2, 16, 16
_NW = _NC * _NS
_PER_W = _N_IN // _NW
_BLK_IN = 12_800
_NBLK = _PER_W // _BLK_IN
_CHUNKS = _BLK_IN // _L
_BLK_OUT = _BLK_IN * _DIM
_PAT = _DIM * _L


def _make_kernel():
    mesh = plsc.VectorSubcoreMesh(core_axis_name="c", subcore_axis_name="s")

    @functools.partial(
        pl.kernel,
        mesh=mesh,
        out_type=jax.ShapeDtypeStruct((_N_IN * _DIM,), jnp.float32),
        compiler_params=pltpu.CompilerParams(needs_layout_passes=False),
        scratch_types=[
            pltpu.VMEM((_PAT,), jnp.int32),
            pltpu.VMEM((_PAT,), jnp.float32),
            pltpu.VMEM((_PAT,), jnp.float32),
            pltpu.VMEM((_BLK_IN,), jnp.int32),
            pltpu.VMEM((_BLK_OUT,), jnp.float32),
        ],
    )
    def emb_kernel(x_hbm, gpat_hbm, w0_hbm, w1_hbm, out_hbm,
                   gp_v, w0_v, w1_v, xb, ob):
        wid = lax.axis_index("s") * _NC + lax.axis_index("c")
        in_base = wid * _PER_W
        out_base = in_base * _DIM

        pltpu.sync_copy(gpat_hbm, gp_v)
        pltpu.sync_copy(w0_hbm, w0_v)
        pltpu.sync_copy(w1_hbm, w1_v)

        def blk_body(b, carry):
            ib = pl.multiple_of(in_base + b * _BLK_IN, 8)
            pltpu.sync_copy(x_hbm.at[pl.ds(ib, _BLK_IN)], xb)

            def chunk(t, c2):
                xoff = t * _L
                for p in range(_DIM):
                    gi = gp_v[pl.ds(p * _L, _L)] + xoff
                    xg = plsc.load_gather(xb, [gi])
                    y = jnp.where(xg > 0,
                                  w1_v[pl.ds(p * _L, _L)],
                                  w0_v[pl.ds(p * _L, _L)])
                    ob[pl.ds(xoff * _DIM + p * _L, _L)] = y
                return c2

            lax.fori_loop(0, _CHUNKS, chunk, 0)

            ob_off = pl.multiple_of(out_base + b * _BLK_OUT, 8)
            pltpu.sync_copy(ob, out_hbm.at[pl.ds(ob_off, _BLK_OUT)])
            return carry

        lax.fori_loop(0, _NBLK, blk_body, 0)

    return emb_kernel


_emb_kernel = _make_kernel()


def kernel(x, emb_weight):
    lane = jnp.arange(_PAT, dtype=jnp.int32)
    gpat = lane // _DIM
    ws = emb_weight.astype(jnp.float32) * jnp.float32(3.0) ** jnp.float32(0.5)
    w0pat = ws[0, lane % _DIM]
    w1pat = ws[1, lane % _DIM]
    out_flat = _emb_kernel(x.reshape(-1), gpat, w0pat, w1pat)
    return out_flat.reshape(_ROWS, _COLS, _DIM)


# speedup vs baseline: 93.1455x; 17.4636x over previous
import functools

import jax
import jax.numpy as jnp
from jax import lax
from jax.experimental import pallas as pl
from jax.experimental.pallas import tpu as pltpu
from jax.experimental.pallas import tpu_sc as plsc

_ROWS, _COLS, _DIM = 16384, 200, 3
_N_IN = _ROWS * _COLS
_NC, _NS, _L = 2, 16, 16
_NW = _NC * _NS
_PER_W = _N_IN // _NW
_BLK = 12_800
_NBLK = _PER_W // _BLK
_CHUNKS = _BLK // _L


def _make_kernel():
    mesh = plsc.VectorSubcoreMesh(core_axis_name="c", subcore_axis_name="s")

    @functools.partial(
        pl.kernel,
        mesh=mesh,
        out_type=jax.ShapeDtypeStruct((_DIM * _N_IN,), jnp.float32),
        compiler_params=pltpu.CompilerParams(needs_layout_passes=False),
        scratch_types=[
            pltpu.VMEM((2 * _DIM * _L,), jnp.float32),
            pltpu.VMEM((_BLK,), jnp.int32),
            pltpu.VMEM((_BLK,), jnp.float32),
            pltpu.VMEM((_BLK,), jnp.float32),
            pltpu.VMEM((_BLK,), jnp.float32),
        ],
    )
    def emb_kernel(x_hbm, wsplat_hbm, out_hbm, ws_v, xb, ob0, ob1, ob2):
        obs = (ob0, ob1, ob2)
        wid = lax.axis_index("s") * _NC + lax.axis_index("c")
        base = wid * _PER_W

        pltpu.sync_copy(wsplat_hbm, ws_v)

        def blk_body(b, carry):
            ib = pl.multiple_of(base + b * _BLK, 8)
            pltpu.sync_copy(x_hbm.at[pl.ds(ib, _BLK)], xb)

            def chunk(t, ws):
                (w00, w01, w02, w10, w11, w12) = ws
                w0 = (w00, w01, w02)
                w1 = (w10, w11, w12)
                off = t * _L
                m = xb[pl.ds(off, _L)] > 0
                for k in range(_DIM):
                    obs[k][pl.ds(off, _L)] = jnp.where(m, w1[k], w0[k])
                return ws

            ws0 = tuple(ws_v[pl.ds(v * _L, _L)] for v in range(2 * _DIM))
            lax.fori_loop(0, _CHUNKS, chunk, ws0, unroll=4)

            for k in range(_DIM):
                ob_off = pl.multiple_of(k * _N_IN + base + b * _BLK, 8)
                pltpu.sync_copy(obs[k], out_hbm.at[pl.ds(ob_off, _BLK)])
            return carry

        lax.fori_loop(0, _NBLK, blk_body, 0)

    return emb_kernel


_emb_kernel = _make_kernel()


def kernel(x, emb_weight):
    ws = emb_weight.astype(jnp.float32) * jnp.float32(3.0) ** jnp.float32(0.5)
    wsplat = jnp.broadcast_to(ws.reshape(2 * _DIM, 1), (2 * _DIM, _L)).reshape(-1)
    xt_flat = x.T.reshape(-1)
    out_flat = _emb_kernel(xt_flat, wsplat)
    return out_flat.reshape(_DIM, _COLS, _ROWS).transpose(2, 1, 0)


# speedup vs baseline: 159.1922x; 1.7091x over previous
import functools

import jax
import jax.numpy as jnp
from jax import lax
from jax.experimental import pallas as pl
from jax.experimental.pallas import tpu as pltpu
from jax.experimental.pallas import tpu_sc as plsc

_ROWS, _COLS, _DIM = 16384, 200, 3
_N_IN = _ROWS * _COLS
_NC, _NS, _L = 2, 16, 16
_NW = _NC * _NS
_PER_W = _N_IN // _NW
_BLK = 12_800
_NBLK = _PER_W // _BLK
_CHUNKS = _BLK // _L
_TS, _TL = 8, 128


def _make_kernel():
    mesh = plsc.VectorSubcoreMesh(core_axis_name="c", subcore_axis_name="s")

    @functools.partial(
        pl.kernel,
        mesh=mesh,
        out_type=jax.ShapeDtypeStruct((_DIM * _N_IN,), jnp.float32),
        compiler_params=pltpu.CompilerParams(needs_layout_passes=False),
        scratch_types=[
            pltpu.VMEM((2 * _DIM * _L,), jnp.float32),
            pltpu.VMEM((_BLK,), jnp.int32),
            pltpu.VMEM((_BLK,), jnp.float32),
            pltpu.VMEM((_BLK,), jnp.float32),
            pltpu.VMEM((_BLK,), jnp.float32),
        ],
    )
    def emb_kernel(x_hbm, wsplat_hbm, out_hbm, ws_v, xb, ob0, ob1, ob2):
        obs = (ob0, ob1, ob2)
        wid = lax.axis_index("s") * _NC + lax.axis_index("c")
        base = wid * _PER_W

        pltpu.sync_copy(wsplat_hbm, ws_v)

        def blk_body(b, carry):
            ib = pl.multiple_of(base + b * _BLK, 8)
            pltpu.sync_copy(x_hbm.at[pl.ds(ib, _BLK)], xb)

            def chunk(t, ws):
                (w00, w01, w02, w10, w11, w12) = ws
                w0 = (w00, w01, w02)
                w1 = (w10, w11, w12)
                off = t * _L
                m = xb[pl.ds(off, _L)] > 0
                for k in range(_DIM):
                    obs[k][pl.ds(off, _L)] = jnp.where(m, w1[k], w0[k])
                return ws

            ws0 = tuple(ws_v[pl.ds(v * _L, _L)] for v in range(2 * _DIM))
            lax.fori_loop(0, _CHUNKS, chunk, ws0, unroll=4)

            for k in range(_DIM):
                ob_off = pl.multiple_of(k * _N_IN + base + b * _BLK, 8)
                pltpu.sync_copy(obs[k], out_hbm.at[pl.ds(ob_off, _BLK)])
            return carry

        lax.fori_loop(0, _NBLK, blk_body, 0)

    return emb_kernel


_emb_kernel = _make_kernel()


def kernel(x, emb_weight):
    ws = emb_weight.astype(jnp.float32) * jnp.float32(3.0) ** jnp.float32(0.5)
    wsplat = jnp.broadcast_to(ws.reshape(2 * _DIM, 1), (2 * _DIM, _L)).reshape(-1)
    xraw = x.reshape(_ROWS // _TL, _TL, _COLS // _TS, _TS)
    xraw = xraw.transpose(2, 0, 3, 1).reshape(-1)
    out_flat = _emb_kernel(xraw, wsplat)
    o5 = out_flat.reshape(_DIM, _COLS // _TS, _ROWS // _TL, _TS, _TL)
    return o5.transpose(2, 4, 1, 3, 0).reshape(_ROWS, _COLS, _DIM)


# speedup vs baseline: 208.5206x; 1.3099x over previous
import functools

import jax
import jax.numpy as jnp
from jax import lax
from jax.experimental import pallas as pl
from jax.experimental.pallas import tpu as pltpu
from jax.experimental.pallas import tpu_sc as plsc

_ROWS, _COLS, _DIM = 16384, 200, 3
_N_IN = _ROWS * _COLS
_NC, _NS, _L = 2, 16, 16
_NW = _NC * _NS
_PER_W = _N_IN // _NW
_BLK = 12_800
_NBLK = _PER_W // _BLK
_CHUNKS = _BLK // _L
_TS, _TL = 8, 128


def _make_kernel():
    mesh = plsc.VectorSubcoreMesh(core_axis_name="c", subcore_axis_name="s")

    @functools.partial(
        pl.kernel,
        mesh=mesh,
        out_type=jax.ShapeDtypeStruct((_DIM * _N_IN,), jnp.float32),
        compiler_params=pltpu.CompilerParams(needs_layout_passes=False),
        scratch_types=(
            [pltpu.VMEM((2 * _DIM * _L,), jnp.float32)]
            + [pltpu.VMEM((_BLK,), jnp.int32)] * 2
            + [pltpu.VMEM((_BLK,), jnp.float32)] * 6
            + [pltpu.SemaphoreType.DMA] * 4
        ),
    )
    def emb_kernel(x_hbm, wsplat_hbm, out_hbm, ws_v,
                   xb0, xb1, oa0, oa1, oa2, ob0, ob1, ob2,
                   sin0, sin1, sout0, sout1):
        xbs = (xb0, xb1)
        obs = ((oa0, oa1, oa2), (ob0, ob1, ob2))
        sins = (sin0, sin1)
        souts = (sout0, sout1)
        wid = lax.axis_index("s") * _NC + lax.axis_index("c")
        base = wid * _PER_W

        pltpu.sync_copy(wsplat_hbm, ws_v)
        ws0 = tuple(ws_v[pl.ds(v * _L, _L)] for v in range(2 * _DIM))

        def start_in(b):
            ib = pl.multiple_of(base + b * _BLK, 8)
            return pltpu.async_copy(
                x_hbm.at[pl.ds(ib, _BLK)], xbs[b % 2], sins[b % 2])

        in_copies = [None] * _NBLK
        out_copies = [None] * _NBLK
        in_copies[0] = start_in(0)
        for b in range(_NBLK):
            buf = b % 2
            if b + 1 < _NBLK:
                in_copies[b + 1] = start_in(b + 1)
            in_copies[b].wait()
            if b >= 2:
                for c in out_copies[b - 2]:
                    c.wait()

            xb = xbs[buf]
            ob = obs[buf]

            def chunk(t, ws):
                (w00, w01, w02, w10, w11, w12) = ws
                w0 = (w00, w01, w02)
                w1 = (w10, w11, w12)
                off = t * _L
                m = xb[pl.ds(off, _L)] > 0
                for k in range(_DIM):
                    ob[k][pl.ds(off, _L)] = jnp.where(m, w1[k], w0[k])
                return ws

            lax.fori_loop(0, _CHUNKS, chunk, ws0, unroll=4)

            ocs = []
            for k in range(_DIM):
                oo = pl.multiple_of(k * _N_IN + base + b * _BLK, 8)
                ocs.append(pltpu.async_copy(
                    ob[k], out_hbm.at[pl.ds(oo, _BLK)], souts[buf]))
            out_copies[b] = ocs
        for b in (_NBLK - 2, _NBLK - 1):
            for c in out_copies[b]:
                c.wait()

    return emb_kernel


_emb_kernel = _make_kernel()


def kernel(x, emb_weight):
    ws = emb_weight.astype(jnp.float32) * jnp.float32(3.0) ** jnp.float32(0.5)
    wsplat = jnp.broadcast_to(ws.reshape(2 * _DIM, 1), (2 * _DIM, _L)).reshape(-1)
    xraw = x.reshape(_ROWS // _TL, _TL, _COLS // _TS, _TS)
    xraw = xraw.transpose(2, 0, 3, 1).reshape(-1)
    out_flat = _emb_kernel(xraw, wsplat)
    o5 = out_flat.reshape(_DIM, _COLS // _TS, _ROWS // _TL, _TS, _TL)
    return o5.transpose(2, 4, 1, 3, 0).reshape(_ROWS, _COLS, _DIM)
